# baseline (speedup 1.0000x reference)
import functools

import jax
import jax.numpy as jnp
from jax import lax
from jax.experimental import pallas as pl
from jax.experimental.pallas import tpu as pltpu
from jax.experimental.pallas import tpu_sc as plsc

VOCAB = 100000
VOCAB_PAD = 102400
DIM = 128
SCALING = 1.0
NPAD = 1024
NC, NS, L = 2, 16, 16
NW = NC * NS
CHUNK = 128

_RID_SLICE = VOCAB_PAD // NS


def _wid():
    return lax.axis_index("s") * NC + lax.axis_index("c")


def _mesh():
    return plsc.VectorSubcoreMesh(core_axis_name="c", subcore_axis_name="s")


_CP = pltpu.CompilerParams(needs_layout_passes=False, use_tc_tiling_on_sc=False)


def _make_fused(n_tok):
    b_per_w = n_tok // NW
    n_chunks = b_per_w // CHUNK

    @functools.partial(
        pl.kernel,
        out_type=jax.ShapeDtypeStruct((n_tok, DIM), jnp.float32),
        mesh=_mesh(),
        compiler_params=_CP,
        scratch_types=[
            pltpu.VMEM((n_chunks, CHUNK), jnp.int32),
            pltpu.VMEM((CHUNK,), jnp.int32),
            pltpu.VMEM((CHUNK,), jnp.int32),
            pltpu.VMEM((CHUNK, DIM), jnp.float32),
            pltpu.VMEM((CHUNK, DIM), jnp.float32),
            pltpu.VMEM((NPAD,), jnp.int32),
            pltpu.VMEM((NPAD,), jnp.int32),
            pltpu.VMEM((NPAD,), jnp.float32),
            pltpu.VMEM((_RID_SLICE,), jnp.int32),
            pltpu.SemaphoreType.DMA,
            pltpu.SemaphoreType.DMA,
            pltpu.SemaphoreType.DMA,
            pltpu.SemaphoreType.DMA,
            pltpu.SemaphoreType.DMA,
            pltpu.SemaphoreType.DMA,
            pltpu.VMEM_SHARED((VOCAB_PAD,), jnp.int32),
        ],
    )
    def fused(xf2_hbm, w_hbm, row_hbm, col_hbm, val_hbm, out_hbm,
              idx2_v, ridv0, ridv1, rows0, rows1, row_v, col_v, val_v,
              rid_loc, sw0, sw1, sr0, sr1, so0, so1, rid_sh):
        wid = _wid()
        s = lax.axis_index("s")
        rows = (rows0, rows1)
        ridv = (ridv0, ridv1)
        sw = (sw0, sw1)
        sr = (sr0, sr1)
        so = (so0, so1)
        out_base = wid * b_per_w

        pltpu.sync_copy(xf2_hbm.at[pl.ds(wid * n_chunks, n_chunks)], idx2_v)
        for b in range(2):
            pltpu.async_copy(w_hbm.at[idx2_v.at[b]], rows[b], sw[b])

        lo = s * _RID_SLICE
        iota16 = lax.iota(jnp.int32, L)

        def zero_body(i, c):
            rid_loc[pl.ds(i * L, L)] = jnp.zeros((L,), jnp.int32)
            return c

        lax.fori_loop(0, _RID_SLICE // L, zero_body, 0)
        pltpu.sync_copy(row_hbm, row_v)
        pltpu.sync_copy(col_hbm, col_v)
        pltpu.sync_copy(val_hbm, val_v)

        def scat_body(i, c):
            off = i * L
            r = row_v[pl.ds(off, L)] - lo
            msk = (r >= 0) & (r < _RID_SLICE)
            plsc.addupdate_scatter(rid_loc, [r], iota16 + (off + 2048), mask=msk)
            return c

        lax.fori_loop(0, NPAD // L, scat_body, 0)

        def ver_body(i, c):
            off = i * L
            r = row_v[pl.ds(off, L)] - lo
            msk = (r >= 0) & (r < _RID_SLICE)
            rc = jnp.minimum(jnp.maximum(r, 0), _RID_SLICE - 1)
            rv = plsc.load_gather(rid_loc, [rc], mask=msk)
            bad = msk & (rv != iota16 + (off + 2048))
            plsc.store_scatter(rid_loc, [rc], jnp.full((L,), 4096, jnp.int32),
                               mask=bad)
            return c

        lax.fori_loop(0, NPAD // L, ver_body, 0)
        pltpu.sync_copy(rid_loc, rid_sh.at[pl.ds(lo, _RID_SLICE)])
        plsc.subcore_barrier()

        def start_gather(t, b):
            @pl.when(t < n_chunks)
            def _():
                @pl.when(t >= 2)
                def _():
                    pltpu.make_async_copy(
                        rows[b], out_hbm.at[pl.ds(out_base, CHUNK)], so[b]
                    ).wait()
                pltpu.async_copy(w_hbm.at[idx2_v.at[t]], rows[b], sw[b])
                pltpu.async_copy(rid_sh.at[idx2_v.at[t]], ridv[b], sr[b])

        def finish_chunk(t, b):
            @pl.when(jnp.logical_and(t >= 0, t < n_chunks))
            def _():
                pltpu.make_async_copy(
                    w_hbm.at[idx2_v.at[0]], rows[b], sw[b]
                ).wait()
                pltpu.make_async_copy(
                    rid_sh.at[idx2_v.at[0]], ridv[b], sr[b]
                ).wait()
                for v in range(CHUNK // L):
                    rv = ridv[b][pl.ds(v * L, L)]
                    cnt = lax.shift_right_logical(rv, 11)
                    lanes = iota16 + (v * L)
                    mx = jnp.max(cnt)

                    @pl.when(mx > 0)
                    def _(rv=rv, cnt=cnt, lanes=lanes, b=b):
                        m1 = cnt == 1
                        j = jnp.minimum(jnp.maximum(rv - 2048, 0), NPAD - 1)
                        col = plsc.load_gather(col_v, [j], mask=m1)
                        val = plsc.load_gather(val_v, [j], mask=m1)
                        plsc.addupdate_scatter(rows[b], [lanes, col], val,
                                               mask=m1)

                    @pl.when(mx >= 2)
                    def _(rv=rv, cnt=cnt, lanes=lanes, b=b, t=t):
                        tok = idx2_v[t, pl.ds(v * L, L)]
                        m2 = cnt >= 2

                        def slow_body(k, cc, tok=tok, m2=m2, lanes=lanes, b=b):
                            kk = jnp.zeros((L,), jnp.int32) + k
                            r_f = plsc.load_gather(row_v, [kk])
                            c_f = plsc.load_gather(col_v, [kk])
                            v_f = plsc.load_gather(val_v, [kk])
                            hit = m2 & (tok == r_f)
                            plsc.addupdate_scatter(rows[b], [lanes, c_f], v_f,
                                                   mask=hit)
                            return cc

                        lax.fori_loop(0, NPAD, slow_body, 0)

                pltpu.async_copy(
                    rows[b], out_hbm.at[pl.ds(out_base + t * CHUNK, CHUNK)], so[b]
                )

        pltpu.async_copy(rid_sh.at[idx2_v.at[0]], ridv0, sr0)
        pltpu.async_copy(rid_sh.at[idx2_v.at[1]], ridv1, sr1)
        finish_chunk(0, 0)

        def turn(i, c):
            for b in range(2):
                t = 2 * i + b
                start_gather(t, b)
                finish_chunk(t - 1, 1 - b)
            return c

        lax.fori_loop(1, (n_chunks + 2) // 2, turn, 0)
        pltpu.make_async_copy(rows0, out_hbm.at[pl.ds(out_base, CHUNK)], so0).wait()
        pltpu.make_async_copy(rows1, out_hbm.at[pl.ds(out_base, CHUNK)], so1).wait()

    return fused


def kernel(x, W, spectrum, row_idx, col_idx):
    bsz, hist = x.shape
    n_tok = bsz * hist
    xf = jnp.transpose(x).reshape(n_tok).astype(jnp.int32)
    n_freq = row_idx.shape[0]

    pad = NPAD - n_freq
    row_p = jnp.pad(row_idx.astype(jnp.int32), (0, pad), constant_values=VOCAB_PAD)
    col_p = jnp.pad(col_idx.astype(jnp.int32), (0, pad))
    val_p = jnp.pad(spectrum.astype(jnp.float32) * SCALING, (0, pad))

    xf2 = xf.reshape(n_tok // CHUNK, CHUNK)
    out = _make_fused(n_tok)(xf2, W, row_p, col_p, val_p)
    return jnp.transpose(out.reshape(hist, bsz, DIM), (1, 0, 2))


# speedup vs baseline: 1.1004x; 1.1004x over previous
import functools

import jax
import jax.numpy as jnp
from jax import lax
from jax.experimental import pallas as pl
from jax.experimental.pallas import tpu as pltpu
from jax.experimental.pallas import tpu_sc as plsc

VOCAB = 100000
VOCAB_PAD = 102400
DIM = 128
SCALING = 1.0
NPAD = 1024
NC, NS, L = 2, 16, 16
NW = NC * NS
CHUNK = 128

_RID_SLICE = VOCAB_PAD // NS


def _wid():
    return lax.axis_index("s") * NC + lax.axis_index("c")


def _mesh():
    return plsc.VectorSubcoreMesh(core_axis_name="c", subcore_axis_name="s")


_CP = pltpu.CompilerParams(needs_layout_passes=False, use_tc_tiling_on_sc=False)


def _make_fused(n_tok):
    b_per_w = n_tok // NW
    n_chunks = b_per_w // CHUNK

    @functools.partial(
        pl.kernel,
        out_type=jax.ShapeDtypeStruct((n_tok, DIM), jnp.float32),
        mesh=_mesh(),
        compiler_params=_CP,
        scratch_types=[
            pltpu.VMEM((n_chunks, CHUNK), jnp.int32),
            pltpu.VMEM((CHUNK,), jnp.int32),
            pltpu.VMEM((CHUNK,), jnp.int32),
            pltpu.VMEM((CHUNK, DIM), jnp.float32),
            pltpu.VMEM((CHUNK, DIM), jnp.float32),
            pltpu.VMEM((NPAD,), jnp.int32),
            pltpu.VMEM((NPAD,), jnp.float32),
            pltpu.VMEM((_RID_SLICE,), jnp.int32),
            pltpu.VMEM((NPAD,), jnp.int32),
            pltpu.VMEM((NPAD,), jnp.int32),
            pltpu.VMEM((CHUNK,), jnp.int32),
            pltpu.VMEM((CHUNK,), jnp.int32),
            pltpu.VMEM((CHUNK,), jnp.float32),
            pltpu.SemaphoreType.DMA,
            pltpu.SemaphoreType.DMA,
            pltpu.SemaphoreType.DMA,
            pltpu.SemaphoreType.DMA,
            pltpu.SemaphoreType.DMA,
            pltpu.SemaphoreType.DMA,
            pltpu.SemaphoreType.DMA,
            pltpu.SemaphoreType.DMA,
            pltpu.VMEM_SHARED((VOCAB_PAD,), jnp.int32),
            pltpu.VMEM_SHARED((NPAD,), jnp.int32),
            pltpu.VMEM_SHARED((NPAD,), jnp.float32),
        ],
    )
    def fused(xf2_hbm, w_hbm, srow_hbm, pval_hbm, order_hbm, col_hbm, val_hbm,
              out_hbm,
              idx2_v, ridv0, ridv1, rows0, rows1, scol_v, sval_v,
              rid_loc, srow_v, pval_v, ord_v, colg_v, valg_v,
              sw0, sw1, sr0, sr1, so0, so1, sg0, sg1,
              rid_sh, scol_sh, sval_sh):
        wid = _wid()
        s = lax.axis_index("s")
        rows = (rows0, rows1)
        ridv = (ridv0, ridv1)
        sw = (sw0, sw1)
        sr = (sr0, sr1)
        so = (so0, so1)
        out_base = wid * b_per_w

        pltpu.sync_copy(xf2_hbm.at[pl.ds(wid * n_chunks, n_chunks)], idx2_v)
        for b in range(2):
            pltpu.async_copy(w_hbm.at[idx2_v.at[b]], rows[b], sw[b])

        lo = s * _RID_SLICE

        def zero_body(i, c):
            rid_loc[pl.ds(i * L, L)] = jnp.zeros((L,), jnp.int32)
            return c

        lax.fori_loop(0, _RID_SLICE // L, zero_body, 0)
        pltpu.sync_copy(srow_hbm, srow_v)
        pltpu.sync_copy(pval_hbm, pval_v)

        def scat_body(i, c):
            off = i * L
            r = srow_v[pl.ds(off, L)] - lo
            v = pval_v[pl.ds(off, L)]
            msk = (r >= 0) & (r < _RID_SLICE)
            plsc.store_scatter(rid_loc, [r], v, mask=msk)
            return c

        lax.fori_loop(0, NPAD // L, scat_body, 0)
        pltpu.sync_copy(rid_loc, rid_sh.at[pl.ds(lo, _RID_SLICE)])

        @pl.when(s < NPAD // CHUNK)
        def _():
            pltpu.sync_copy(order_hbm.at[pl.ds(s * CHUNK, CHUNK)], ord_v)
            pltpu.async_copy(col_hbm.at[ord_v], colg_v, sg0)
            pltpu.async_copy(val_hbm.at[ord_v], valg_v, sg1)
            pltpu.make_async_copy(col_hbm.at[ord_v], colg_v, sg0).wait()
            pltpu.make_async_copy(val_hbm.at[ord_v], valg_v, sg1).wait()
            pltpu.sync_copy(colg_v, scol_sh.at[pl.ds(s * CHUNK, CHUNK)])
            pltpu.sync_copy(valg_v, sval_sh.at[pl.ds(s * CHUNK, CHUNK)])

        plsc.subcore_barrier()
        pltpu.sync_copy(scol_sh, scol_v)
        pltpu.sync_copy(sval_sh, sval_v)

        def start_gather(t, b):
            @pl.when(t < n_chunks)
            def _():
                @pl.when(t >= 2)
                def _():
                    pltpu.make_async_copy(
                        rows[b], out_hbm.at[pl.ds(out_base, CHUNK)], so[b]
                    ).wait()
                pltpu.async_copy(w_hbm.at[idx2_v.at[t]], rows[b], sw[b])
                pltpu.async_copy(rid_sh.at[idx2_v.at[t]], ridv[b], sr[b])

        def finish_chunk(t, b):
            @pl.when(jnp.logical_and(t >= 0, t < n_chunks))
            def _():
                pltpu.make_async_copy(
                    w_hbm.at[idx2_v.at[0]], rows[b], sw[b]
                ).wait()
                pltpu.make_async_copy(
                    rid_sh.at[idx2_v.at[0]], ridv[b], sr[b]
                ).wait()
                for v in range(CHUNK // L):
                    rv = ridv[b][pl.ds(v * L, L)]
                    cnt = lax.bitwise_and(rv, 2047)
                    start = lax.shift_right_logical(rv, 11)
                    m = jnp.max(cnt)
                    lanes = lax.iota(jnp.int32, L) + (v * L)

                    def delta_body(k, cc, start=start, cnt=cnt, lanes=lanes, b=b):
                        msk = k < cnt
                        j = jnp.minimum(start + k, NPAD - 1)
                        col = plsc.load_gather(scol_v, [j], mask=msk)
                        val = plsc.load_gather(sval_v, [j], mask=msk)
                        plsc.addupdate_scatter(rows[b], [lanes, col], val, mask=msk)
                        return cc

                    lax.fori_loop(0, m, delta_body, 0)
                pltpu.async_copy(
                    rows[b], out_hbm.at[pl.ds(out_base + t * CHUNK, CHUNK)], so[b]
                )

        pltpu.async_copy(rid_sh.at[idx2_v.at[0]], ridv0, sr0)
        pltpu.async_copy(rid_sh.at[idx2_v.at[1]], ridv1, sr1)
        finish_chunk(0, 0)

        def turn(i, c):
            for b in range(2):
                t = 2 * i + b
                start_gather(t, b)
                finish_chunk(t - 1, 1 - b)
            return c

        lax.fori_loop(1, (n_chunks + 2) // 2, turn, 0)
        pltpu.make_async_copy(rows0, out_hbm.at[pl.ds(out_base, CHUNK)], so0).wait()
        pltpu.make_async_copy(rows1, out_hbm.at[pl.ds(out_base, CHUNK)], so1).wait()

    return fused


def kernel(x, W, spectrum, row_idx, col_idx):
    bsz, hist = x.shape
    n_tok = bsz * hist
    xf = jnp.transpose(x).reshape(n_tok).astype(jnp.int32)
    n_freq = row_idx.shape[0]

    pad = NPAD - n_freq
    iota = jnp.arange(NPAD, dtype=jnp.int32)
    row_p = jnp.pad(row_idx.astype(jnp.int32), (0, pad), constant_values=VOCAB_PAD)
    srow_p, order_p = lax.sort_key_val(row_p, iota)
    is_start = jnp.concatenate([jnp.ones((1,), bool), srow_p[1:] != srow_p[:-1]])
    first = lax.cummax(jnp.where(is_start, iota, 0))
    is_end = jnp.concatenate([srow_p[:-1] != srow_p[1:], jnp.ones((1,), bool)])
    last = jnp.flip(lax.cummin(jnp.flip(jnp.where(is_end, iota, NPAD - 1))))
    packed_p = first * 2048 + (last - first + 1)
    col_p = jnp.pad(col_idx.astype(jnp.int32), (0, pad))
    val_p = jnp.pad(spectrum.astype(jnp.float32) * SCALING, (0, pad))

    xf2 = xf.reshape(n_tok // CHUNK, CHUNK)
    out = _make_fused(n_tok)(xf2, W, srow_p, packed_p, order_p, col_p, val_p)
    return jnp.transpose(out.reshape(hist, bsz, DIM), (1, 0, 2))
